# Initial kernel scaffold; baseline (speedup 1.0000x reference)
import jax
import jax.numpy as jnp
from jax.experimental import pallas as pl


def kernel(x, emb_weight):
    raise NotImplementedError("write your pallas kernel here")


# speedup vs baseline: 5.3456x; 5.3456x over previous
import functools

import jax
import jax.numpy as jnp
from jax import lax
from jax.experimental import pallas as pl
from jax.experimental.pallas import tpu as pltpu
from jax.experimental.pallas import tpu_sc as plsc

_GATHER_DNUMS = lax.GatherDimensionNumbers(
    offset_dims=(), collapsed_slice_dims=(0,), start_index_map=(0,)
)


def _xlane_gather(vec, idx):
    return lax.gather(
        vec,
        idx[:, None],
        dimension_numbers=_GATHER_DNUMS,
        slice_sizes=(1,),
        mode=lax.GatherScatterMode.PROMISE_IN_BOUNDS,
    )


@functools.partial(jax.jit, static_argnames=("n", "per_w", "sub", "iters"))
def _lookup_flat(x_flat, w_flat, *, n, per_w, sub, iters):
    mesh = plsc.VectorSubcoreMesh(core_axis_name="c", subcore_axis_name="s")
    info = plsc.get_sparse_core_info()
    nc = info.num_cores
    n_grp = sub // 16

    @functools.partial(
        pl.kernel,
        mesh=mesh,
        out_type=jax.ShapeDtypeStruct((4 * n,), jnp.float32),
        scratch_types=[
            pltpu.VMEM((2, sub), jnp.int32),
            pltpu.VMEM((2, 4 * sub), jnp.float32),
            pltpu.VMEM((16,), jnp.float32),
            pltpu.SemaphoreType.DMA,
            pltpu.SemaphoreType.DMA,
            pltpu.SemaphoreType.DMA,
        ],
    )
    def k(x_hbm, w_hbm, out_hbm, x_v, out_v, tbl_v, in_sem, out_sem0, out_sem1):
        wid = lax.axis_index("s") * nc + lax.axis_index("c")
        base = wid * per_w

        pltpu.sync_copy(w_hbm, tbl_v)
        t = tbl_v[...]
        io = lax.iota(jnp.int32, 16)
        r = lax.bitwise_and(io, 3)
        q = lax.shift_right_logical(io, 2)
        w_rows = [_xlane_gather(t, 4 * v + r) for v in range(4)]
        perms = [q + 4 * m for m in range(4)]
        out_sems = (out_sem0, out_sem1)

        def start_in(it, slot):
            pltpu.async_copy(
                x_hbm.at[pl.ds(base + it * sub, sub)], x_v.at[slot], in_sem
            )

        def wait_in(slot):
            pltpu.make_async_copy(
                x_hbm.at[pl.ds(0, sub)], x_v.at[slot], in_sem
            ).wait()

        def wait_out(slot):
            pltpu.make_async_copy(
                out_v.at[slot], out_hbm.at[pl.ds(0, 4 * sub)], out_sems[slot]
            ).wait()

        start_in(0, 0)

        def pair_body(it2, _):
            for slot in (0, 1):
                it = 2 * it2 + slot
                wait_in(slot)

                @pl.when(it + 1 < iters)
                def _():
                    start_in(it + 1, 1 - slot)

                @pl.when(it2 >= 1)
                def _():
                    wait_out(slot)

                def grp(g, _):
                    xv = x_v[slot, pl.ds(g * 16, 16)]
                    for m in range(4):
                        xg = _xlane_gather(xv, perms[m])
                        o = jnp.where(
                            xg == 0,
                            w_rows[0],
                            jnp.where(
                                xg == 1,
                                w_rows[1],
                                jnp.where(xg == 2, w_rows[2], w_rows[3]),
                            ),
                        )
                        out_v[slot, pl.ds(g * 64 + m * 16, 16)] = o
                    return 0

                lax.fori_loop(0, n_grp, grp, 0, unroll=2)

                pltpu.async_copy(
                    out_v.at[slot],
                    out_hbm.at[pl.ds(4 * (base + it * sub), 4 * sub)],
                    out_sems[slot],
                )
            return 0

        lax.fori_loop(0, iters // 2, pair_body, 0)
        wait_out(0)
        wait_out(1)

    return k(x_flat, w_flat)


def kernel(x, emb_weight):
    b, t = x.shape
    n = b * t
    x_flat = x.reshape(n).astype(jnp.int32)
    w_flat = emb_weight.reshape(16).astype(jnp.float32)

    nw = 32
    per_w = n // nw
    assert per_w * nw == n
    sub = max(
        c
        for c in range(16, 12801, 16)
        if per_w % c == 0 and (per_w // c) % 2 == 0
    )
    iters = per_w // sub

    y_flat = _lookup_flat(x_flat, w_flat, n=n, per_w=per_w, sub=sub, iters=iters)
    return y_flat.reshape(b, t, 4)


# speedup vs baseline: 5.3669x; 1.0040x over previous
import functools

import jax
import jax.numpy as jnp
from jax import lax
from jax.experimental import pallas as pl
from jax.experimental.pallas import tpu as pltpu
from jax.experimental.pallas import tpu_sc as plsc

_GATHER_DNUMS = lax.GatherDimensionNumbers(
    offset_dims=(), collapsed_slice_dims=(0,), start_index_map=(0,)
)


def _xlane_gather(vec, idx):
    return lax.gather(
        vec,
        idx[:, None],
        dimension_numbers=_GATHER_DNUMS,
        slice_sizes=(1,),
        mode=lax.GatherScatterMode.PROMISE_IN_BOUNDS,
    )


@functools.partial(jax.jit, static_argnames=("n", "per_w", "sub", "iters"))
def _lookup_flat(x_flat, w_flat, *, n, per_w, sub, iters):
    mesh = plsc.VectorSubcoreMesh(core_axis_name="c", subcore_axis_name="s")
    info = plsc.get_sparse_core_info()
    nc = info.num_cores
    n_grp = sub // 16

    @functools.partial(
        pl.kernel,
        mesh=mesh,
        out_type=jax.ShapeDtypeStruct((4 * n,), jnp.float32),
        scratch_types=[
            pltpu.VMEM((2, sub), jnp.int32),
            pltpu.VMEM((2, 4 * sub), jnp.float32),
            pltpu.VMEM((16,), jnp.float32),
            pltpu.SemaphoreType.DMA,
            pltpu.SemaphoreType.DMA,
            pltpu.SemaphoreType.DMA,
        ],
    )
    def k(x_hbm, w_hbm, out_hbm, x_v, out_v, tbl_v, in_sem, out_sem0, out_sem1):
        wid = lax.axis_index("s") * nc + lax.axis_index("c")
        base = wid * per_w

        pltpu.sync_copy(w_hbm, tbl_v)
        t = tbl_v[...]
        io = lax.iota(jnp.int32, 16)
        r = lax.bitwise_and(io, 3)
        q = lax.shift_right_logical(io, 2)
        w_rows = [_xlane_gather(t, 4 * v + r) for v in range(4)]
        perms = [q + 4 * m for m in range(4)]
        out_sems = (out_sem0, out_sem1)

        def start_in(it, slot):
            pltpu.async_copy(
                x_hbm.at[pl.ds(base + it * sub, sub)], x_v.at[slot], in_sem
            )

        def wait_in(slot):
            pltpu.make_async_copy(
                x_hbm.at[pl.ds(0, sub)], x_v.at[slot], in_sem
            ).wait()

        def wait_out(slot):
            pltpu.make_async_copy(
                out_v.at[slot], out_hbm.at[pl.ds(0, 4 * sub)], out_sems[slot]
            ).wait()

        start_in(0, 0)

        def pair_body(it2, _):
            for slot in (0, 1):
                it = 2 * it2 + slot
                wait_in(slot)

                @pl.when(it + 1 < iters)
                def _():
                    start_in(it + 1, 1 - slot)

                @pl.when(it2 >= 1)
                def _():
                    wait_out(slot)

                def grp(g, _):
                    xv = x_v[slot, pl.ds(g * 16, 16)]
                    for m in range(4):
                        xg = _xlane_gather(xv, perms[m])
                        o = jnp.where(
                            xg == 0,
                            w_rows[0],
                            jnp.where(
                                xg == 1,
                                w_rows[1],
                                jnp.where(xg == 2, w_rows[2], w_rows[3]),
                            ),
                        )
                        out_v[slot, pl.ds(g * 64 + m * 16, 16)] = o
                    return 0

                lax.fori_loop(0, n_grp, grp, 0, unroll=8)

                pltpu.async_copy(
                    out_v.at[slot],
                    out_hbm.at[pl.ds(4 * (base + it * sub), 4 * sub)],
                    out_sems[slot],
                )
            return 0

        lax.fori_loop(0, iters // 2, pair_body, 0)
        wait_out(0)
        wait_out(1)

    return k(x_flat, w_flat)


def kernel(x, emb_weight):
    b, t = x.shape
    n = b * t
    x_flat = x.reshape(n).astype(jnp.int32)
    w_flat = emb_weight.reshape(16).astype(jnp.float32)

    nw = 32
    per_w = n // nw
    assert per_w * nw == n
    sub = max(
        c
        for c in range(16, 12801, 16)
        if per_w % c == 0 and (per_w // c) % 2 == 0
    )
    iters = per_w // sub

    y_flat = _lookup_flat(x_flat, w_flat, n=n, per_w=per_w, sub=sub, iters=iters)
    return y_flat.reshape(b, t, 4)
